# Initial kernel scaffold; baseline (speedup 1.0000x reference)
import functools

import jax
import jax.numpy as jnp
from jax import lax
from jax.experimental import pallas as pl
from jax.experimental.pallas import tpu as pltpu
from jax.experimental.pallas import tpu_sc as plsc

N = 10000
E = 160000
D = 256

NC = 2
NS = 16
W = 80
EPT = E // NS
NWIN = EPT // W
ROWW = 144
NPT = N // NS


def _proj_body(x_ref, w_ref, b_ref, q_ref, k_ref, v_ref, xr_ref):
    y = jnp.dot(x_ref[...], w_ref[...], preferred_element_type=jnp.float32)
    y = y + b_ref[...]
    q_ref[0] = y[:, 0:128]
    q_ref[1] = y[:, 128:256]
    k_ref[0] = y[:, 256:384]
    k_ref[1] = y[:, 384:512]
    v_ref[0] = y[:, 512:640]
    v_ref[1] = y[:, 640:768]
    xr_ref[...] = y[:, 768:1024]


def _project(x, wall, ball):
    bn = 1000
    grid = (N // bn,)
    out_shapes = (
        jax.ShapeDtypeStruct((2, N, 128), jnp.float32),
        jax.ShapeDtypeStruct((2, N, 128), jnp.float32),
        jax.ShapeDtypeStruct((2, N, 128), jnp.float32),
        jax.ShapeDtypeStruct((N, 256), jnp.float32),
    )
    pair_spec = pl.BlockSpec((2, bn, 128), lambda i: (0, i, 0))
    return pl.pallas_call(
        _proj_body,
        grid=grid,
        in_specs=[
            pl.BlockSpec((bn, D), lambda i: (i, 0)),
            pl.BlockSpec((D, 1024), lambda i: (0, 0)),
            pl.BlockSpec((1, 1024), lambda i: (0, 0)),
        ],
        out_specs=(pair_spec, pair_spec, pair_spec,
                   pl.BlockSpec((bn, 256), lambda i: (i, 0))),
        out_shape=out_shapes,
    )(x, wall, ball)


def _sc_edge_kernel(qcat, kcat, vcat, src, dst):
    mesh = plsc.VectorSubcoreMesh(core_axis_name="c", subcore_axis_name="s")

    @functools.partial(
        pl.kernel,
        out_type=jax.ShapeDtypeStruct((2, N, ROWW), jnp.float32),
        mesh=mesh,
        scratch_types=[
            pltpu.VMEM((W,), jnp.int32),
            pltpu.VMEM((W,), jnp.int32),
            pltpu.VMEM((W,), jnp.int32),
            pltpu.VMEM((W,), jnp.int32),
            pltpu.VMEM((W, 128), jnp.float32),
            pltpu.VMEM((W, 128), jnp.float32),
            pltpu.VMEM((W, 128), jnp.float32),
            pltpu.VMEM((W, ROWW), jnp.float32),
            pltpu.VMEM_SHARED((N, ROWW), jnp.float32),
            pltpu.SemaphoreType.DMA,
            pltpu.SemaphoreType.DMA,
            pltpu.SemaphoreType.DMA,
        ],
    )
    def k(q_hbm, k_hbm, v_hbm, src_hbm, dst_hbm, out_hbm,
          srci, dsti, srca, dsta, qd, ks, vs, msg, acc, sem1, sem2, sem3):
        cid = lax.axis_index("c")
        sid = lax.axis_index("s")
        zero = jnp.zeros((16,), jnp.float32)
        lanes = lax.iota(jnp.int32, 16)
        m0 = jnp.where(lanes == 0, 1.0, 0.0).astype(jnp.float32)
        m1 = jnp.where(lanes == 1, 1.0, 0.0).astype(jnp.float32)
        coff = cid * N

        @pl.loop(0, W)
        def _(e):
            @pl.loop(0, ROWW, step=16)
            def _(j):
                msg[e, pl.ds(j, 16)] = zero

        node_base = sid * NPT

        @pl.loop(0, NPT // W)
        def _(i):
            pltpu.sync_copy(msg, acc.at[pl.ds(node_base + i * W, W)])

        rem = NPT - (NPT // W) * W
        pltpu.sync_copy(msg.at[pl.ds(0, rem)],
                        acc.at[pl.ds(node_base + (NPT // W) * W, rem)])
        plsc.subcore_barrier()

        edge_base = sid * EPT

        @pl.loop(0, NWIN)
        def _(w):
            eb = edge_base + w * W
            pltpu.sync_copy(src_hbm.at[pl.ds(eb, W)], srci)
            pltpu.sync_copy(dst_hbm.at[pl.ds(eb, W)], dsti)

            @pl.loop(0, W, step=16)
            def _(j):
                srca[pl.ds(j, 16)] = srci[pl.ds(j, 16)] + coff
                dsta[pl.ds(j, 16)] = dsti[pl.ds(j, 16)] + coff

            cp1 = pltpu.async_copy(q_hbm.at[dsta], qd, sem1)
            cp2 = pltpu.async_copy(k_hbm.at[srca], ks, sem2)
            cp3 = pltpu.async_copy(v_hbm.at[srca], vs, sem3)
            cp1.wait()
            cp2.wait()
            cp3.wait()

            @pl.loop(0, W)
            def _(e):
                p0 = qd[e, pl.ds(0, 16)] * ks[e, pl.ds(0, 16)]
                p0 = p0 + qd[e, pl.ds(16, 16)] * ks[e, pl.ds(16, 16)]
                p0 = p0 + qd[e, pl.ds(32, 16)] * ks[e, pl.ds(32, 16)]
                p0 = p0 + qd[e, pl.ds(48, 16)] * ks[e, pl.ds(48, 16)]
                a0 = jnp.sum(p0) * 0.125
                p1 = qd[e, pl.ds(64, 16)] * ks[e, pl.ds(64, 16)]
                p1 = p1 + qd[e, pl.ds(80, 16)] * ks[e, pl.ds(80, 16)]
                p1 = p1 + qd[e, pl.ds(96, 16)] * ks[e, pl.ds(96, 16)]
                p1 = p1 + qd[e, pl.ds(112, 16)] * ks[e, pl.ds(112, 16)]
                a1 = jnp.sum(p1) * 0.125
                e0 = jnp.exp(jnp.full((16,), a0, jnp.float32))
                e1 = jnp.exp(jnp.full((16,), a1, jnp.float32))
                msg[e, pl.ds(0, 16)] = vs[e, pl.ds(0, 16)] * e0
                msg[e, pl.ds(16, 16)] = vs[e, pl.ds(16, 16)] * e0
                msg[e, pl.ds(32, 16)] = vs[e, pl.ds(32, 16)] * e0
                msg[e, pl.ds(48, 16)] = vs[e, pl.ds(48, 16)] * e0
                msg[e, pl.ds(64, 16)] = vs[e, pl.ds(64, 16)] * e1
                msg[e, pl.ds(80, 16)] = vs[e, pl.ds(80, 16)] * e1
                msg[e, pl.ds(96, 16)] = vs[e, pl.ds(96, 16)] * e1
                msg[e, pl.ds(112, 16)] = vs[e, pl.ds(112, 16)] * e1
                msg[e, pl.ds(128, 16)] = e0 * m0 + e1 * m1

            pltpu.sync_copy(msg, acc.at[dsti], add=True)

        plsc.subcore_barrier()
        pltpu.sync_copy(acc.at[pl.ds(node_base, NPT)],
                        out_hbm.at[cid, pl.ds(node_base, NPT)])

    return k(qcat, kcat, vcat, src, dst)


def _fin_body(acc_ref, xr_ref, wa_ref, wb_ref, o_ref):
    a0 = acc_ref[0]
    a1 = acc_ref[1]
    msg = jnp.concatenate([a0[:, 0:128], a1[:, 0:128]], axis=1)
    den4 = jnp.concatenate(
        [a0[:, 128:129], a0[:, 129:130], a1[:, 128:129], a1[:, 129:130]], axis=1)
    sel = (jax.lax.broadcasted_iota(jnp.int32, (4, 256), 1) // 64
           == jax.lax.broadcasted_iota(jnp.int32, (4, 256), 0)).astype(jnp.float32)
    den = jnp.dot(den4, sel, preferred_element_type=jnp.float32) + 1e-16
    out = msg / den
    xr = xr_ref[...]
    s = (jnp.dot(out, wa_ref[...], preferred_element_type=jnp.float32)
         + jnp.dot(xr, wb_ref[...], preferred_element_type=jnp.float32))
    g = jax.nn.sigmoid(s)
    o_ref[...] = g * xr + (1.0 - g) * out


def _finalize(acc, xr, wa, wb):
    bn = 1000
    return pl.pallas_call(
        _fin_body,
        grid=(N // bn,),
        in_specs=[
            pl.BlockSpec((2, bn, ROWW), lambda i: (0, i, 0)),
            pl.BlockSpec((bn, 256), lambda i: (i, 0)),
            pl.BlockSpec((256, 1), lambda i: (0, 0)),
            pl.BlockSpec((256, 1), lambda i: (0, 0)),
        ],
        out_specs=pl.BlockSpec((bn, 256), lambda i: (i, 0)),
        out_shape=jax.ShapeDtypeStruct((N, 256), jnp.float32),
    )(acc, xr, wa, wb)


def kernel(x, edge_index, Wq, bq, Wk, bk, Wv, bv, Wskip, bskip, Wbeta):
    wall = jnp.concatenate([Wq, Wk, Wv, Wskip], axis=1)
    ball = jnp.concatenate([bq, bk, bv, bskip]).reshape(1, 1024)
    qh, kh, vh, xr = _project(x, wall, ball)
    qcat = qh.reshape(2 * N, 128)
    kcat = kh.reshape(2 * N, 128)
    vcat = vh.reshape(2 * N, 128)
    src = edge_index[0]
    dst = edge_index[1]
    acc = _sc_edge_kernel(qcat, kcat, vcat, src, dst)
    wa = Wbeta[0:256] + Wbeta[512:768]
    wb = Wbeta[256:512] - Wbeta[512:768]
    return _finalize(acc, xr, wa, wb)


# speedup vs baseline: 16.9796x; 16.9796x over previous
import dataclasses
import functools

import jax
import jax.numpy as jnp
from jax import lax
from jax.experimental import pallas as pl
from jax.experimental.pallas import tpu as pltpu
from jax.experimental.pallas import tpu_sc as plsc

N = 10000
E = 160000
D = 256

NC = 2
NS = 16
W = 80
EPT = E // NS
NWIN = EPT // W
ROWW = 136
NSTRIDE = 624
NSPAN = 640


def _proj_body(x_ref, w_ref, b_ref, q_ref, k_ref, v_ref, xr_ref):
    y = jnp.dot(x_ref[...], w_ref[...], preferred_element_type=jnp.float32)
    y = y + b_ref[...]
    q_ref[0] = y[:, 0:128]
    q_ref[1] = y[:, 128:256]
    k_ref[0] = y[:, 256:384]
    k_ref[1] = y[:, 384:512]
    v_ref[0] = y[:, 512:640]
    v_ref[1] = y[:, 640:768]
    xr_ref[...] = y[:, 768:1024]


def _project(x, wall, ball):
    bn = 1000
    grid = (N // bn,)
    out_shapes = (
        jax.ShapeDtypeStruct((2, N, 128), jnp.float32),
        jax.ShapeDtypeStruct((2, N, 128), jnp.float32),
        jax.ShapeDtypeStruct((2, N, 128), jnp.float32),
        jax.ShapeDtypeStruct((N, 256), jnp.float32),
    )
    pair_spec = pl.BlockSpec((2, bn, 128), lambda i: (0, i, 0))
    return pl.pallas_call(
        _proj_body,
        grid=grid,
        in_specs=[
            pl.BlockSpec((bn, D), lambda i: (i, 0)),
            pl.BlockSpec((D, 1024), lambda i: (0, 0)),
            pl.BlockSpec((1, 1024), lambda i: (0, 0)),
        ],
        out_specs=(pair_spec, pair_spec, pair_spec,
                   pl.BlockSpec((bn, 256), lambda i: (i, 0))),
        out_shape=out_shapes,
    )(x, wall, ball)


def _sc_edge_kernel(qcat, kcat, vcat, src, dst):
    mesh = plsc.VectorSubcoreMesh(core_axis_name="c", subcore_axis_name="s")
    cp = pltpu.CompilerParams()
    if "needs_layout_passes" in pltpu.CompilerParams.__dataclass_fields__:
        cp = dataclasses.replace(cp, needs_layout_passes=False)
    if "use_tc_tiling_on_sc" in pltpu.CompilerParams.__dataclass_fields__:
        cp = dataclasses.replace(cp, use_tc_tiling_on_sc=False)

    @functools.partial(
        pl.kernel,
        out_type=jax.ShapeDtypeStruct((2, N, ROWW), jnp.float32),
        mesh=mesh,
        compiler_params=cp,
        scratch_types=[
            pltpu.VMEM((W,), jnp.int32),
            pltpu.VMEM((W,), jnp.int32),
            pltpu.VMEM((W,), jnp.int32),
            pltpu.VMEM((W,), jnp.int32),
            pltpu.VMEM((W, 128), jnp.float32),
            pltpu.VMEM((W, 128), jnp.float32),
            pltpu.VMEM((W, 128), jnp.float32),
            pltpu.VMEM((W, ROWW), jnp.float32),
            pltpu.VMEM_SHARED((N, ROWW), jnp.float32),
            pltpu.SemaphoreType.DMA,
            pltpu.SemaphoreType.DMA,
            pltpu.SemaphoreType.DMA,
        ],
    )
    def k(q_hbm, k_hbm, v_hbm, src_hbm, dst_hbm, out_hbm,
          srci, dsti, srca, dsta, qd, ks, vs, msg, acc, sem1, sem2, sem3):
        cid = lax.axis_index("c")
        sid = lax.axis_index("s")
        zero = jnp.zeros((16,), jnp.float32)
        lanes = lax.iota(jnp.int32, 16)
        m8 = jnp.where(lanes == 8, 1.0, 0.0).astype(jnp.float32)
        m9 = jnp.where(lanes == 9, 1.0, 0.0).astype(jnp.float32)
        mlow = jnp.where(lanes < 8, 1.0, 0.0).astype(jnp.float32)
        coff = cid * N

        @pl.loop(0, W)
        def _(e):
            @pl.loop(0, 128, step=16)
            def _(j):
                msg[e, pl.ds(j, 16)] = zero

            msg[e, pl.ds(120, 16)] = zero

        node_base = sid * NSTRIDE

        @pl.loop(0, NSPAN // W)
        def _(i):
            pltpu.sync_copy(msg, acc.at[pl.ds(node_base + i * W, W)])

        plsc.subcore_barrier()

        edge_base = sid * EPT

        @pl.loop(0, NWIN)
        def _(w):
            eb = edge_base + w * W
            pltpu.sync_copy(src_hbm.at[pl.ds(eb, W)], srci)
            pltpu.sync_copy(dst_hbm.at[pl.ds(eb, W)], dsti)

            @pl.loop(0, W, step=16)
            def _(j):
                srca[pl.ds(j, 16)] = srci[pl.ds(j, 16)] + coff
                dsta[pl.ds(j, 16)] = dsti[pl.ds(j, 16)] + coff

            cp1 = pltpu.async_copy(q_hbm.at[dsta], qd, sem1)
            cp2 = pltpu.async_copy(k_hbm.at[srca], ks, sem2)
            cp3 = pltpu.async_copy(v_hbm.at[srca], vs, sem3)
            cp1.wait()
            cp2.wait()
            cp3.wait()

            @pl.loop(0, W)
            def _(e):
                p0 = qd[e, pl.ds(0, 16)] * ks[e, pl.ds(0, 16)]
                p0 = p0 + qd[e, pl.ds(16, 16)] * ks[e, pl.ds(16, 16)]
                p0 = p0 + qd[e, pl.ds(32, 16)] * ks[e, pl.ds(32, 16)]
                p0 = p0 + qd[e, pl.ds(48, 16)] * ks[e, pl.ds(48, 16)]
                a0 = jnp.sum(p0) * 0.125
                p1 = qd[e, pl.ds(64, 16)] * ks[e, pl.ds(64, 16)]
                p1 = p1 + qd[e, pl.ds(80, 16)] * ks[e, pl.ds(80, 16)]
                p1 = p1 + qd[e, pl.ds(96, 16)] * ks[e, pl.ds(96, 16)]
                p1 = p1 + qd[e, pl.ds(112, 16)] * ks[e, pl.ds(112, 16)]
                a1 = jnp.sum(p1) * 0.125
                e0 = jnp.exp(jnp.full((16,), a0, jnp.float32))
                e1 = jnp.exp(jnp.full((16,), a1, jnp.float32))
                msg[e, pl.ds(0, 16)] = vs[e, pl.ds(0, 16)] * e0
                msg[e, pl.ds(16, 16)] = vs[e, pl.ds(16, 16)] * e0
                msg[e, pl.ds(32, 16)] = vs[e, pl.ds(32, 16)] * e0
                msg[e, pl.ds(48, 16)] = vs[e, pl.ds(48, 16)] * e0
                msg[e, pl.ds(64, 16)] = vs[e, pl.ds(64, 16)] * e1
                msg[e, pl.ds(80, 16)] = vs[e, pl.ds(80, 16)] * e1
                msg[e, pl.ds(96, 16)] = vs[e, pl.ds(96, 16)] * e1
                msg[e, pl.ds(112, 16)] = vs[e, pl.ds(112, 16)] * e1
                t = msg[e, pl.ds(120, 16)]
                msg[e, pl.ds(120, 16)] = t * mlow + e0 * m8 + e1 * m9

            pltpu.sync_copy(msg, acc.at[dsti], add=True)

        plsc.subcore_barrier()
        pltpu.sync_copy(acc.at[pl.ds(node_base, NSPAN)],
                        out_hbm.at[cid, pl.ds(node_base, NSPAN)])

    return k(qcat, kcat, vcat, src, dst)


def _fin_body(acc_ref, xr_ref, wa_ref, wb_ref, o_ref):
    a0 = acc_ref[0]
    a1 = acc_ref[1]
    msg = jnp.concatenate([a0[:, 0:128], a1[:, 0:128]], axis=1)
    den4 = jnp.concatenate(
        [a0[:, 128:129], a0[:, 129:130], a1[:, 128:129], a1[:, 129:130]], axis=1)
    sel = (jax.lax.broadcasted_iota(jnp.int32, (4, 256), 1) // 64
           == jax.lax.broadcasted_iota(jnp.int32, (4, 256), 0)).astype(jnp.float32)
    den = jnp.dot(den4, sel, preferred_element_type=jnp.float32) + 1e-16
    out = msg / den
    xr = xr_ref[...]
    s = (jnp.dot(out, wa_ref[...], preferred_element_type=jnp.float32)
         + jnp.dot(xr, wb_ref[...], preferred_element_type=jnp.float32))
    g = jax.nn.sigmoid(s)
    o_ref[...] = g * xr + (1.0 - g) * out


def _finalize(acc, xr, wa, wb):
    bn = 1000
    return pl.pallas_call(
        _fin_body,
        grid=(N // bn,),
        in_specs=[
            pl.BlockSpec((2, bn, ROWW), lambda i: (0, i, 0)),
            pl.BlockSpec((bn, 256), lambda i: (i, 0)),
            pl.BlockSpec((256, 1), lambda i: (0, 0)),
            pl.BlockSpec((256, 1), lambda i: (0, 0)),
        ],
        out_specs=pl.BlockSpec((bn, 256), lambda i: (i, 0)),
        out_shape=jax.ShapeDtypeStruct((N, 256), jnp.float32),
    )(acc, xr, wa, wb)


def kernel(x, edge_index, Wq, bq, Wk, bk, Wv, bv, Wskip, bskip, Wbeta):
    wall = jnp.concatenate([Wq, Wk, Wv, Wskip], axis=1)
    ball = jnp.concatenate([bq, bk, bv, bskip]).reshape(1, 1024)
    qh, kh, vh, xr = _project(x, wall, ball)
    qcat = qh.reshape(2 * N, 128)
    kcat = kh.reshape(2 * N, 128)
    vcat = vh.reshape(2 * N, 128)
    src = edge_index[0]
    dst = edge_index[1]
    acc = _sc_edge_kernel(qcat, kcat, vcat, src, dst)
    wa = Wbeta[0:256] + Wbeta[512:768]
    wb = Wbeta[256:512] - Wbeta[512:768]
    return _finalize(acc, xr, wa, wb)


# speedup vs baseline: 18.8054x; 1.1075x over previous
import dataclasses
import functools

import jax
import jax.numpy as jnp
from jax import lax
from jax.experimental import pallas as pl
from jax.experimental.pallas import tpu as pltpu
from jax.experimental.pallas import tpu_sc as plsc

N = 10000
E = 160000
D = 256

NC = 2
NS = 16
W = 40
EPT = E // NS
NWIN = EPT // W
NPAIR = NWIN // 2
ROWW = 136
NSTRIDE = 624
NSPAN = 640


def _proj_body(x_ref, w_ref, b_ref, q_ref, k_ref, v_ref, xr_ref):
    y = jnp.dot(x_ref[...], w_ref[...], preferred_element_type=jnp.float32)
    y = y + b_ref[...]
    q_ref[0] = y[:, 0:128]
    q_ref[1] = y[:, 128:256]
    k_ref[0] = y[:, 256:384]
    k_ref[1] = y[:, 384:512]
    v_ref[0] = y[:, 512:640]
    v_ref[1] = y[:, 640:768]
    xr_ref[...] = y[:, 768:1024]


def _project(x, wall, ball):
    bn = 1000
    grid = (N // bn,)
    out_shapes = (
        jax.ShapeDtypeStruct((2, N, 128), jnp.float32),
        jax.ShapeDtypeStruct((2, N, 128), jnp.float32),
        jax.ShapeDtypeStruct((2, N, 128), jnp.float32),
        jax.ShapeDtypeStruct((N, 256), jnp.float32),
    )
    pair_spec = pl.BlockSpec((2, bn, 128), lambda i: (0, i, 0))
    return pl.pallas_call(
        _proj_body,
        grid=grid,
        in_specs=[
            pl.BlockSpec((bn, D), lambda i: (i, 0)),
            pl.BlockSpec((D, 1024), lambda i: (0, 0)),
            pl.BlockSpec((1, 1024), lambda i: (0, 0)),
        ],
        out_specs=(pair_spec, pair_spec, pair_spec,
                   pl.BlockSpec((bn, 256), lambda i: (i, 0))),
        out_shape=out_shapes,
    )(x, wall, ball)


def _sc_edge_kernel(qcat, kcat, vcat, src, dst):
    mesh = plsc.VectorSubcoreMesh(core_axis_name="c", subcore_axis_name="s")
    cp = pltpu.CompilerParams()
    if "needs_layout_passes" in pltpu.CompilerParams.__dataclass_fields__:
        cp = dataclasses.replace(cp, needs_layout_passes=False)
    if "use_tc_tiling_on_sc" in pltpu.CompilerParams.__dataclass_fields__:
        cp = dataclasses.replace(cp, use_tc_tiling_on_sc=False)

    idx_t = pltpu.VMEM((W,), jnp.int32)
    row_t = pltpu.VMEM((W, 128), jnp.float32)
    msg_t = pltpu.VMEM((W, ROWW), jnp.float32)
    bufset = [idx_t, idx_t, idx_t, idx_t, idx_t, row_t, row_t, row_t, msg_t]

    @functools.partial(
        pl.kernel,
        out_type=jax.ShapeDtypeStruct((2, N, ROWW), jnp.float32),
        mesh=mesh,
        compiler_params=cp,
        scratch_types=bufset + bufset + [
            pltpu.VMEM_SHARED((N, ROWW), jnp.float32),
        ] + [pltpu.SemaphoreType.DMA] * 10,
    )
    def k(q_hbm, k_hbm, v_hbm, src_hbm, dst_hbm, out_hbm,
          srciA, dstiA, srcaA, dstaA, sdxA, qdA, ksA, vsA, msgA,
          srciB, dstiB, srcaB, dstaB, sdxB, qdB, ksB, vsB, msgB,
          acc, s_ia, s_ib, s_gaq, s_gak, s_gav, s_gbq, s_gbk, s_gbv,
          s_sa, s_sb):
        cid = lax.axis_index("c")
        sid = lax.axis_index("s")
        zero = jnp.zeros((16,), jnp.float32)
        lanes = lax.iota(jnp.int32, 16)
        m8 = jnp.where(lanes == 8, 1.0, 0.0).astype(jnp.float32)
        m9 = jnp.where(lanes == 9, 1.0, 0.0).astype(jnp.float32)
        mlow = jnp.where(lanes < 8, 1.0, 0.0).astype(jnp.float32)
        coff = cid * N
        edge_base = sid * EPT
        node_base = sid * NSTRIDE

        def adjust(srci, dsti, srca, dsta):
            @pl.loop(0, W - 16, step=16)
            def _(j):
                srca[pl.ds(j, 16)] = srci[pl.ds(j, 16)] + coff
                dsta[pl.ds(j, 16)] = dsti[pl.ds(j, 16)] + coff

            srca[pl.ds(W - 16, 16)] = srci[pl.ds(W - 16, 16)] + coff
            dsta[pl.ds(W - 16, 16)] = dsti[pl.ds(W - 16, 16)] + coff

        def snapshot_dst(dsti, sdx):
            @pl.loop(0, W - 16, step=16)
            def _(j):
                sdx[pl.ds(j, 16)] = dsti[pl.ds(j, 16)]

            sdx[pl.ds(W - 16, 16)] = dsti[pl.ds(W - 16, 16)]

        def load_idx(w, srci, dsti, sem):
            eb = edge_base + w * W
            pltpu.sync_copy(src_hbm.at[pl.ds(eb, W)], srci)
            pltpu.sync_copy(dst_hbm.at[pl.ds(eb, W)], dsti)

        def wait_idx(srci, dsti, sem):
            pass

        def start_gathers(srca, dsta, qd, ks, vs, sq, sk, sv):
            pltpu.async_copy(q_hbm.at[dsta], qd, sq)
            pltpu.async_copy(k_hbm.at[srca], ks, sk)
            pltpu.async_copy(v_hbm.at[srca], vs, sv)

        def wait_gathers(srca, dsta, qd, ks, vs, sq, sk, sv):
            pltpu.make_async_copy(q_hbm.at[dsta], qd, sq).wait()
            pltpu.make_async_copy(k_hbm.at[srca], ks, sk).wait()
            pltpu.make_async_copy(v_hbm.at[srca], vs, sv).wait()

        def compute(qd, ks, vs, msg):
            @pl.loop(0, W)
            def _(e):
                p0 = qd[e, pl.ds(0, 16)] * ks[e, pl.ds(0, 16)]
                p0 = p0 + qd[e, pl.ds(16, 16)] * ks[e, pl.ds(16, 16)]
                p0 = p0 + qd[e, pl.ds(32, 16)] * ks[e, pl.ds(32, 16)]
                p0 = p0 + qd[e, pl.ds(48, 16)] * ks[e, pl.ds(48, 16)]
                a0 = jnp.sum(p0) * 0.125
                p1 = qd[e, pl.ds(64, 16)] * ks[e, pl.ds(64, 16)]
                p1 = p1 + qd[e, pl.ds(80, 16)] * ks[e, pl.ds(80, 16)]
                p1 = p1 + qd[e, pl.ds(96, 16)] * ks[e, pl.ds(96, 16)]
                p1 = p1 + qd[e, pl.ds(112, 16)] * ks[e, pl.ds(112, 16)]
                a1 = jnp.sum(p1) * 0.125
                e0 = jnp.exp(jnp.full((16,), a0, jnp.float32))
                e1 = jnp.exp(jnp.full((16,), a1, jnp.float32))
                msg[e, pl.ds(0, 16)] = vs[e, pl.ds(0, 16)] * e0
                msg[e, pl.ds(16, 16)] = vs[e, pl.ds(16, 16)] * e0
                msg[e, pl.ds(32, 16)] = vs[e, pl.ds(32, 16)] * e0
                msg[e, pl.ds(48, 16)] = vs[e, pl.ds(48, 16)] * e0
                msg[e, pl.ds(64, 16)] = vs[e, pl.ds(64, 16)] * e1
                msg[e, pl.ds(80, 16)] = vs[e, pl.ds(80, 16)] * e1
                msg[e, pl.ds(96, 16)] = vs[e, pl.ds(96, 16)] * e1
                msg[e, pl.ds(112, 16)] = vs[e, pl.ds(112, 16)] * e1
                t = msg[e, pl.ds(120, 16)]
                msg[e, pl.ds(120, 16)] = t * mlow + e0 * m8 + e1 * m9

        def start_scatter(msg, sdx, sem):
            pltpu.sync_copy(msg, acc.at[sdx], add=True)

        def wait_scatter(msg, sdx, sem):
            pass

        @pl.loop(0, W)
        def _(e):
            @pl.loop(0, 128, step=16)
            def _(j):
                msgA[e, pl.ds(j, 16)] = zero

            msgA[e, pl.ds(120, 16)] = zero

        @pl.loop(0, NSPAN // W)
        def _(i):
            pltpu.sync_copy(msgA, acc.at[pl.ds(node_base + i * W, W)])

        plsc.subcore_barrier()

        pltpu.sync_copy(src_hbm.at[pl.ds(edge_base, W)], srciA)
        pltpu.sync_copy(dst_hbm.at[pl.ds(edge_base, W)], dstiA)
        adjust(srciA, dstiA, srcaA, dstaA)
        start_gathers(srcaA, dstaA, qdA, ksA, vsA, s_gaq, s_gak, s_gav)
        load_idx(1, srciB, dstiB, s_ib)

        @pl.loop(0, NPAIR)
        def _(i):
            w = 2 * i
            wnext_a = jnp.minimum(w + 2, NWIN - 2)
            wnext_b = jnp.minimum(w + 3, NWIN - 1)
            wait_idx(srciB, dstiB, s_ib)
            adjust(srciB, dstiB, srcaB, dstaB)
            start_gathers(srcaB, dstaB, qdB, ksB, vsB, s_gbq, s_gbk, s_gbv)
            wait_gathers(srcaA, dstaA, qdA, ksA, vsA, s_gaq, s_gak, s_gav)
            compute(qdA, ksA, vsA, msgA)
            pltpu.sync_copy(msgA, acc.at[dstiA], add=True)
            load_idx(wnext_a, srciA, dstiA, s_ia)
            adjust(srciA, dstiA, srcaA, dstaA)
            start_gathers(srcaA, dstaA, qdA, ksA, vsA, s_gaq, s_gak, s_gav)
            wait_gathers(srcaB, dstaB, qdB, ksB, vsB, s_gbq, s_gbk, s_gbv)
            compute(qdB, ksB, vsB, msgB)
            pltpu.sync_copy(msgB, acc.at[dstiB], add=True)
            load_idx(wnext_b, srciB, dstiB, s_ib)

        wait_gathers(srcaA, dstaA, qdA, ksA, vsA, s_gaq, s_gak, s_gav)
        wait_idx(srciB, dstiB, s_ib)

        plsc.subcore_barrier()
        pltpu.sync_copy(acc.at[pl.ds(node_base, NSPAN)],
                        out_hbm.at[cid, pl.ds(node_base, NSPAN)])

    return k(qcat, kcat, vcat, src, dst)


def _fin_body(acc_ref, xr_ref, wa_ref, wb_ref, o_ref):
    a0 = acc_ref[0]
    a1 = acc_ref[1]
    msg = jnp.concatenate([a0[:, 0:128], a1[:, 0:128]], axis=1)
    den4 = jnp.concatenate(
        [a0[:, 128:129], a0[:, 129:130], a1[:, 128:129], a1[:, 129:130]], axis=1)
    sel = (jax.lax.broadcasted_iota(jnp.int32, (4, 256), 1) // 64
           == jax.lax.broadcasted_iota(jnp.int32, (4, 256), 0)).astype(jnp.float32)
    den = jnp.dot(den4, sel, preferred_element_type=jnp.float32) + 1e-16
    out = msg / den
    xr = xr_ref[...]
    s = (jnp.dot(out, wa_ref[...], preferred_element_type=jnp.float32)
         + jnp.dot(xr, wb_ref[...], preferred_element_type=jnp.float32))
    g = jax.nn.sigmoid(s)
    o_ref[...] = g * xr + (1.0 - g) * out


def _finalize(acc, xr, wa, wb):
    bn = 1000
    return pl.pallas_call(
        _fin_body,
        grid=(N // bn,),
        in_specs=[
            pl.BlockSpec((2, bn, ROWW), lambda i: (0, i, 0)),
            pl.BlockSpec((bn, 256), lambda i: (i, 0)),
            pl.BlockSpec((256, 1), lambda i: (0, 0)),
            pl.BlockSpec((256, 1), lambda i: (0, 0)),
        ],
        out_specs=pl.BlockSpec((bn, 256), lambda i: (i, 0)),
        out_shape=jax.ShapeDtypeStruct((N, 256), jnp.float32),
    )(acc, xr, wa, wb)


def kernel(x, edge_index, Wq, bq, Wk, bk, Wv, bv, Wskip, bskip, Wbeta):
    wall = jnp.concatenate([Wq, Wk, Wv, Wskip], axis=1)
    ball = jnp.concatenate([bq, bk, bv, bskip]).reshape(1, 1024)
    qh, kh, vh, xr = _project(x, wall, ball)
    qcat = qh.reshape(2 * N, 128)
    kcat = kh.reshape(2 * N, 128)
    vcat = vh.reshape(2 * N, 128)
    src = edge_index[0]
    dst = edge_index[1]
    acc = _sc_edge_kernel(qcat, kcat, vcat, src, dst)
    wa = Wbeta[0:256] + Wbeta[512:768]
    wb = Wbeta[256:512] - Wbeta[512:768]
    return _finalize(acc, xr, wa, wb)


# speedup vs baseline: 19.6636x; 1.0456x over previous
import dataclasses
import functools

import jax
import jax.numpy as jnp
from jax import lax
from jax.experimental import pallas as pl
from jax.experimental.pallas import tpu as pltpu
from jax.experimental.pallas import tpu_sc as plsc

N = 10000
E = 160000
D = 256

NC = 2
NS = 16
W = 40
EPT = E // NS
NWIN = EPT // W
NPAIR = NWIN // 2
ROWW = 136
NSTRIDE = 624
NSPAN = 640


def _proj_body(x_ref, w_ref, b_ref, q_ref, k_ref, v_ref, xr_ref):
    y = jnp.dot(x_ref[...], w_ref[...], preferred_element_type=jnp.float32)
    y = y + b_ref[...]
    q_ref[0] = y[:, 0:128]
    q_ref[1] = y[:, 128:256]
    k_ref[0] = y[:, 256:384]
    k_ref[1] = y[:, 384:512]
    v_ref[0] = y[:, 512:640]
    v_ref[1] = y[:, 640:768]
    xr_ref[...] = y[:, 768:1024]


def _project(x, wall, ball):
    bn = 1000
    grid = (N // bn,)
    out_shapes = (
        jax.ShapeDtypeStruct((2, N, 128), jnp.float32),
        jax.ShapeDtypeStruct((2, N, 128), jnp.float32),
        jax.ShapeDtypeStruct((2, N, 128), jnp.float32),
        jax.ShapeDtypeStruct((N, 256), jnp.float32),
    )
    pair_spec = pl.BlockSpec((2, bn, 128), lambda i: (0, i, 0))
    return pl.pallas_call(
        _proj_body,
        grid=grid,
        in_specs=[
            pl.BlockSpec((bn, D), lambda i: (i, 0)),
            pl.BlockSpec((D, 1024), lambda i: (0, 0)),
            pl.BlockSpec((1, 1024), lambda i: (0, 0)),
        ],
        out_specs=(pair_spec, pair_spec, pair_spec,
                   pl.BlockSpec((bn, 256), lambda i: (i, 0))),
        out_shape=out_shapes,
    )(x, wall, ball)


def _sc_edge_kernel(qcat, kcat, vcat, src, dst):
    mesh = plsc.VectorSubcoreMesh(core_axis_name="c", subcore_axis_name="s")
    cp = pltpu.CompilerParams()
    if "needs_layout_passes" in pltpu.CompilerParams.__dataclass_fields__:
        cp = dataclasses.replace(cp, needs_layout_passes=False)
    if "use_tc_tiling_on_sc" in pltpu.CompilerParams.__dataclass_fields__:
        cp = dataclasses.replace(cp, use_tc_tiling_on_sc=False)

    idx_t = pltpu.VMEM((W,), jnp.int32)
    row_t = pltpu.VMEM((W, 128), jnp.float32)
    msg_t = pltpu.VMEM((W, ROWW), jnp.float32)
    bufset = [idx_t, idx_t, idx_t, idx_t, idx_t, row_t, row_t, row_t, msg_t]

    @functools.partial(
        pl.kernel,
        out_type=jax.ShapeDtypeStruct((2, N, ROWW), jnp.float32),
        mesh=mesh,
        compiler_params=cp,
        scratch_types=bufset + bufset + [
            pltpu.VMEM_SHARED((N, ROWW), jnp.float32),
        ] + [pltpu.SemaphoreType.DMA] * 10,
    )
    def k(q_hbm, k_hbm, v_hbm, src_hbm, dst_hbm, out_hbm,
          srciA, dstiA, srcaA, dstaA, sdxA, qdA, ksA, vsA, msgA,
          srciB, dstiB, srcaB, dstaB, sdxB, qdB, ksB, vsB, msgB,
          acc, s_ia, s_ib, s_gaq, s_gak, s_gav, s_gbq, s_gbk, s_gbv,
          s_sa, s_sb):
        cid = lax.axis_index("c")
        sid = lax.axis_index("s")
        zero = jnp.zeros((16,), jnp.float32)
        lanes = lax.iota(jnp.int32, 16)
        m8 = jnp.where(lanes == 8, 1.0, 0.0).astype(jnp.float32)
        m9 = jnp.where(lanes == 9, 1.0, 0.0).astype(jnp.float32)
        mlow = jnp.where(lanes < 8, 1.0, 0.0).astype(jnp.float32)
        shift8 = jnp.minimum(lanes + 8, 15)
        coff = cid * N
        edge_base = sid * EPT
        node_base = sid * NSTRIDE

        def adjust(srci, dsti, srca, dsta):
            @pl.loop(0, W - 16, step=16)
            def _(j):
                srca[pl.ds(j, 16)] = srci[pl.ds(j, 16)] + coff
                dsta[pl.ds(j, 16)] = dsti[pl.ds(j, 16)] + coff

            srca[pl.ds(W - 16, 16)] = srci[pl.ds(W - 16, 16)] + coff
            dsta[pl.ds(W - 16, 16)] = dsti[pl.ds(W - 16, 16)] + coff

        def snapshot_dst(dsti, sdx):
            @pl.loop(0, W - 16, step=16)
            def _(j):
                sdx[pl.ds(j, 16)] = dsti[pl.ds(j, 16)]

            sdx[pl.ds(W - 16, 16)] = dsti[pl.ds(W - 16, 16)]

        def load_idx(w, srci, dsti, sem):
            eb = edge_base + w * W
            pltpu.sync_copy(src_hbm.at[pl.ds(eb, W)], srci)
            pltpu.sync_copy(dst_hbm.at[pl.ds(eb, W)], dsti)

        def wait_idx(srci, dsti, sem):
            pass

        def start_gathers(srca, dsta, qd, ks, vs, sq, sk, sv):
            pltpu.async_copy(q_hbm.at[dsta], qd, sq)
            pltpu.async_copy(k_hbm.at[srca], ks, sk)
            pltpu.async_copy(v_hbm.at[srca], vs, sv)

        def wait_gathers(srca, dsta, qd, ks, vs, sq, sk, sv):
            pltpu.make_async_copy(q_hbm.at[dsta], qd, sq).wait()
            pltpu.make_async_copy(k_hbm.at[srca], ks, sk).wait()
            pltpu.make_async_copy(v_hbm.at[srca], vs, sv).wait()

        def compute(qd, ks, vs, msg):
            @pl.loop(0, W, step=4)
            def _(eb4):
                for o in range(4):
                    e = eb4 + o
                    p0 = qd[e, pl.ds(0, 16)] * ks[e, pl.ds(0, 16)]
                    p0 = p0 + qd[e, pl.ds(16, 16)] * ks[e, pl.ds(16, 16)]
                    p0 = p0 + qd[e, pl.ds(32, 16)] * ks[e, pl.ds(32, 16)]
                    p0 = p0 + qd[e, pl.ds(48, 16)] * ks[e, pl.ds(48, 16)]
                    a0 = jnp.sum(p0)
                    p1 = qd[e, pl.ds(64, 16)] * ks[e, pl.ds(64, 16)]
                    p1 = p1 + qd[e, pl.ds(80, 16)] * ks[e, pl.ds(80, 16)]
                    p1 = p1 + qd[e, pl.ds(96, 16)] * ks[e, pl.ds(96, 16)]
                    p1 = p1 + qd[e, pl.ds(112, 16)] * ks[e, pl.ds(112, 16)]
                    a1 = jnp.sum(p1)
                    e0 = jnp.exp(jnp.full((16,), a0, jnp.float32))
                    e1 = jnp.exp(jnp.full((16,), a1, jnp.float32))
                    msg[e, pl.ds(0, 16)] = vs[e, pl.ds(0, 16)] * e0
                    msg[e, pl.ds(16, 16)] = vs[e, pl.ds(16, 16)] * e0
                    msg[e, pl.ds(32, 16)] = vs[e, pl.ds(32, 16)] * e0
                    msg[e, pl.ds(48, 16)] = vs[e, pl.ds(48, 16)] * e0
                    msg[e, pl.ds(64, 16)] = vs[e, pl.ds(64, 16)] * e1
                    msg[e, pl.ds(80, 16)] = vs[e, pl.ds(80, 16)] * e1
                    msg[e, pl.ds(96, 16)] = vs[e, pl.ds(96, 16)] * e1
                    v7 = vs[e, pl.ds(112, 16)]
                    msg[e, pl.ds(112, 16)] = v7 * e1
                    tl = lax.gather(
                        v7, shift8[:, None],
                        lax.GatherDimensionNumbers(
                            offset_dims=(), collapsed_slice_dims=(0,),
                            start_index_map=(0,)),
                        slice_sizes=(1,),
                        mode=lax.GatherScatterMode.PROMISE_IN_BOUNDS)
                    msg[e, pl.ds(120, 16)] = tl * (e1 * mlow) + e0 * m8 + e1 * m9

        def start_scatter(msg, sdx, sem):
            pltpu.sync_copy(msg, acc.at[sdx], add=True)

        def wait_scatter(msg, sdx, sem):
            pass

        @pl.loop(0, W)
        def _(e):
            @pl.loop(0, 128, step=16)
            def _(j):
                msgA[e, pl.ds(j, 16)] = zero

            msgA[e, pl.ds(120, 16)] = zero

        @pl.loop(0, NSPAN // W)
        def _(i):
            pltpu.sync_copy(msgA, acc.at[pl.ds(node_base + i * W, W)])

        plsc.subcore_barrier()

        pltpu.sync_copy(src_hbm.at[pl.ds(edge_base, W)], srciA)
        pltpu.sync_copy(dst_hbm.at[pl.ds(edge_base, W)], dstiA)
        adjust(srciA, dstiA, srcaA, dstaA)
        start_gathers(srcaA, dstaA, qdA, ksA, vsA, s_gaq, s_gak, s_gav)
        load_idx(1, srciB, dstiB, s_ib)

        @pl.loop(0, NPAIR)
        def _(i):
            w = 2 * i
            wnext_a = jnp.minimum(w + 2, NWIN - 2)
            wnext_b = jnp.minimum(w + 3, NWIN - 1)
            wait_idx(srciB, dstiB, s_ib)
            adjust(srciB, dstiB, srcaB, dstaB)
            start_gathers(srcaB, dstaB, qdB, ksB, vsB, s_gbq, s_gbk, s_gbv)
            wait_gathers(srcaA, dstaA, qdA, ksA, vsA, s_gaq, s_gak, s_gav)

            @pl.when(i > 0)
            def _():
                wait_scatter(msgA, sdxA, s_sa)

            snapshot_dst(dstiA, sdxA)
            load_idx(wnext_a, srciA, dstiA, s_ia)
            compute(qdA, ksA, vsA, msgA)
            start_scatter(msgA, sdxA, s_sa)
            wait_idx(srciA, dstiA, s_ia)
            adjust(srciA, dstiA, srcaA, dstaA)
            start_gathers(srcaA, dstaA, qdA, ksA, vsA, s_gaq, s_gak, s_gav)
            wait_gathers(srcaB, dstaB, qdB, ksB, vsB, s_gbq, s_gbk, s_gbv)

            @pl.when(i > 0)
            def _():
                wait_scatter(msgB, sdxB, s_sb)

            snapshot_dst(dstiB, sdxB)
            load_idx(wnext_b, srciB, dstiB, s_ib)
            compute(qdB, ksB, vsB, msgB)
            start_scatter(msgB, sdxB, s_sb)

        wait_scatter(msgA, sdxA, s_sa)
        wait_scatter(msgB, sdxB, s_sb)
        wait_gathers(srcaA, dstaA, qdA, ksA, vsA, s_gaq, s_gak, s_gav)
        wait_idx(srciB, dstiB, s_ib)

        plsc.subcore_barrier()
        pltpu.sync_copy(acc.at[pl.ds(node_base, NSPAN)],
                        out_hbm.at[cid, pl.ds(node_base, NSPAN)])

    return k(qcat, kcat, vcat, src, dst)


def _fin_body(acc_ref, xr_ref, wa_ref, wb_ref, o_ref):
    a0 = acc_ref[0]
    a1 = acc_ref[1]
    msg = jnp.concatenate([a0[:, 0:128], a1[:, 0:128]], axis=1)
    den4 = jnp.concatenate(
        [a0[:, 128:129], a0[:, 129:130], a1[:, 128:129], a1[:, 129:130]], axis=1)
    sel = (jax.lax.broadcasted_iota(jnp.int32, (4, 256), 1) // 64
           == jax.lax.broadcasted_iota(jnp.int32, (4, 256), 0)).astype(jnp.float32)
    den = jnp.dot(den4, sel, preferred_element_type=jnp.float32) + 1e-16
    out = msg / den
    xr = xr_ref[...]
    s = (jnp.dot(out, wa_ref[...], preferred_element_type=jnp.float32)
         + jnp.dot(xr, wb_ref[...], preferred_element_type=jnp.float32))
    g = jax.nn.sigmoid(s)
    o_ref[...] = g * xr + (1.0 - g) * out


def _finalize(acc, xr, wa, wb):
    bn = 1000
    return pl.pallas_call(
        _fin_body,
        grid=(N // bn,),
        in_specs=[
            pl.BlockSpec((2, bn, ROWW), lambda i: (0, i, 0)),
            pl.BlockSpec((bn, 256), lambda i: (i, 0)),
            pl.BlockSpec((256, 1), lambda i: (0, 0)),
            pl.BlockSpec((256, 1), lambda i: (0, 0)),
        ],
        out_specs=pl.BlockSpec((bn, 256), lambda i: (i, 0)),
        out_shape=jax.ShapeDtypeStruct((N, 256), jnp.float32),
    )(acc, xr, wa, wb)


def kernel(x, edge_index, Wq, bq, Wk, bk, Wv, bv, Wskip, bskip, Wbeta):
    wall = jnp.concatenate([Wq, Wk * 0.125, Wv, Wskip], axis=1)
    ball = jnp.concatenate([bq, bk * 0.125, bv, bskip]).reshape(1, 1024)
    qh, kh, vh, xr = _project(x, wall, ball)
    qcat = qh.reshape(2 * N, 128)
    kcat = kh.reshape(2 * N, 128)
    vcat = vh.reshape(2 * N, 128)
    src = edge_index[0]
    dst = edge_index[1]
    acc = _sc_edge_kernel(qcat, kcat, vcat, src, dst)
    wa = Wbeta[0:256] + Wbeta[512:768]
    wb = Wbeta[256:512] - Wbeta[512:768]
    return _finalize(acc, xr, wa, wb)
